# baseline (speedup 1.0000x reference)
import jax
import jax.numpy as jnp
from jax import lax
from jax.experimental import pallas as pl
from jax.experimental.pallas import tpu as pltpu
from jax.experimental.pallas import tpu_sc as plsc

_IN = 128
_STEPS = 3
_G = 512
_GP = 528
_NW = 32
_NPAD = 100352
_NPW = _NPAD // _NW
_CHUNK = 112
_NCH = _NPW // _CHUNK
_NEG = -1e30


def _wid():
    return lax.axis_index("s") * 2 + lax.axis_index("c")


def _x_pipeline(x_hbm, x_v0, x_v1, base, semx0, semx1, process):
    def _issue(k, buf, sem):
        off = (base + k * _CHUNK) * _IN
        pltpu.make_async_copy(
            x_hbm.at[pl.ds(off, _CHUNK * _IN)], buf, sem).start()

    def _wait(buf, sem):
        pltpu.make_async_copy(
            x_hbm.at[pl.ds(0, _CHUNK * _IN)], buf, sem).wait()

    _issue(0, x_v0, semx0)

    def _pair(k, _):
        _issue(2 * k + 1, x_v1, semx1)
        _wait(x_v0, semx0)
        process(2 * k, x_v0)

        @pl.when(k < _NCH // 2 - 1)
        def _():
            _issue(2 * k + 2, x_v0, semx0)

        _wait(x_v1, semx1)
        process(2 * k + 1, x_v1)
        return 0
    lax.fori_loop(0, _NCH // 2, _pair, 0)


def _sc_e_kernel(x_hbm, b_hbm, q_hbm, e_hbm, mk_hbm,
                 q_v, x_v0, x_v1, ids_v, e_v, m16_v, mks_v, semx0, semx1):
    wid = _wid()
    base = wid * _NPW
    iota = lax.iota(jnp.int32, 16)

    pltpu.sync_copy(b_hbm.at[pl.ds(base, _NPW)], ids_v)
    pltpu.sync_copy(q_hbm, q_v.at[pl.ds(0, _G * _IN)])

    def _minit(i, _):
        m16_v[pl.ds(i * 16, 16)] = jnp.full((16,), _NEG, jnp.float32)
        return 0
    lax.fori_loop(0, _GP, _minit, 0)

    def _qzero(i, _):
        q_v[pl.ds(_G * _IN + i * 16, 16)] = jnp.zeros((16,), jnp.float32)
        return 0
    lax.fori_loop(0, (_GP - _G) * _IN // 16, _qzero, 0)

    def _process(kk, xbuf):
        off = kk * _CHUNK

        def _grp(g, _):
            nn = off + g * 16
            bvec = ids_v[pl.ds(nn, 16)]

            def _uniform():
                b0 = bvec[0]
                xb = (iota + g * 16) * _IN

                def _cblk(ci, accs):
                    qreg = q_v[pl.ds(b0 * _IN + ci * 16, 16)]
                    out = list(accs)
                    for u in range(16):
                        xc = plsc.load_gather(xbuf, [xb + (ci * 16 + u)])
                        out[u % 8] = out[u % 8] + xc * qreg[u]
                    return tuple(out)
                accs = lax.fori_loop(
                    0, _IN // 16, _cblk,
                    tuple(jnp.zeros((16,), jnp.float32) for _ in range(8)))
                return (((accs[0] + accs[1]) + (accs[2] + accs[3])) +
                        ((accs[4] + accs[5]) + (accs[6] + accs[7])))

            def _mixed():
                xb = (iota + g * 16) * _IN
                qb = bvec * _IN

                def _cblk(ci, accs):
                    out = []
                    for u in range(8):
                        cidx = ci * 8 + u
                        xc = plsc.load_gather(xbuf, [xb + cidx])
                        qc = plsc.load_gather(q_v, [qb + cidx])
                        out = out + [accs[u] + xc * qc]
                    return tuple(out)
                accs = lax.fori_loop(
                    0, _IN // 8, _cblk,
                    tuple(jnp.zeros((16,), jnp.float32) for _ in range(8)))
                return (((accs[0] + accs[1]) + (accs[2] + accs[3])) +
                        ((accs[4] + accs[5]) + (accs[6] + accs[7])))

            acc = lax.cond(bvec[0] == bvec[15], _uniform, _mixed)
            e_v[pl.ds(nn, 16)] = acc
            i16 = bvec * 16 + iota
            mg = plsc.load_gather(m16_v, [i16])
            plsc.store_scatter(m16_v, [i16], jnp.maximum(mg, acc))
            return 0
        lax.fori_loop(0, _CHUNK // 16, _grp, 0)

    _x_pipeline(x_hbm, x_v0, x_v1, base, semx0, semx1, _process)

    def _mred(i, _):
        rows = (iota + i * 16) * 16
        acc = jnp.full((16,), _NEG, jnp.float32)
        for j in range(16):
            acc = jnp.maximum(acc, plsc.load_gather(m16_v, [rows + j]))
        mks_v[pl.ds(i * 16, 16)] = acc
        return 0
    lax.fori_loop(0, _G // 16, _mred, 0)

    pltpu.sync_copy(e_v, e_hbm.at[pl.ds(base, _NPW)])
    pltpu.sync_copy(mks_v, mk_hbm.at[wid])


def _sc_r_kernel(x_hbm, b_hbm, e_hbm, mk_hbm, sk_hbm, rk_hbm,
                 x_v0, x_v1, ids_v, e_v, m_v, s16_v, r_v, mk_v, sks_v,
                 semx0, semx1):
    wid = _wid()
    base = wid * _NPW
    iota = lax.iota(jnp.int32, 16)

    pltpu.sync_copy(mk_hbm, mk_v)
    pltpu.sync_copy(b_hbm.at[pl.ds(base, _NPW)], ids_v)
    pltpu.sync_copy(e_hbm.at[pl.ds(base, _NPW)], e_v)

    def _mred(i, _):
        sl = pl.ds(i * 16, 16)

        def _w(w, a):
            return jnp.maximum(a, mk_v[w, sl])
        m_v[sl] = lax.fori_loop(0, _NW, _w,
                                jnp.full((16,), _NEG, jnp.float32))
        return 0
    lax.fori_loop(0, _G // 16, _mred, 0)

    def _mpad(i, _):
        m_v[pl.ds(_G + i * 16, 16)] = jnp.zeros((16,), jnp.float32)
        return 0
    lax.fori_loop(0, (_GP - _G) // 16, _mpad, 0)

    def _szero(i, _):
        s16_v[pl.ds(i * 16, 16)] = jnp.zeros((16,), jnp.float32)
        return 0
    lax.fori_loop(0, _GP, _szero, 0)

    def _rzero(i, _):
        for j in range(_IN // 16):
            r_v[i, pl.ds(j * 16, 16)] = jnp.zeros((16,), jnp.float32)
        return 0
    lax.fori_loop(0, _GP, _rzero, 0)

    def _process(kk, xbuf):
        off = kk * _CHUNK

        def _grp(g, _):
            nn = off + g * 16
            bvec = ids_v[pl.ds(nn, 16)]
            evec = e_v[pl.ds(nn, 16)]
            mg = plsc.load_gather(m_v, [bvec])
            exv = jnp.exp(evec - mg)
            i16 = bvec * 16 + iota
            sg = plsc.load_gather(s16_v, [i16])
            plsc.store_scatter(s16_v, [i16], sg + exv)
            xo0 = g * 16 * _IN

            def _uniform():
                exns = [exv[l] for l in range(16)]
                accs = [jnp.zeros((16,), jnp.float32) for _ in range(8)]
                for l in range(16):
                    for j in range(_IN // 16):
                        accs[j] = accs[j] + exns[l] * xbuf[
                            pl.ds(xo0 + l * _IN + j * 16, 16)]
                bn = bvec[0]
                for j in range(_IN // 16):
                    sl = pl.ds(j * 16, 16)
                    r_v[bn, sl] = r_v[bn, sl] + accs[j]

            def _mixed():
                for l in range(16):
                    bn = bvec[l]
                    exn = exv[l]
                    xo = xo0 + l * _IN
                    for j in range(_IN // 16):
                        sl = pl.ds(j * 16, 16)
                        r_v[bn, sl] = (r_v[bn, sl] +
                                       exn * xbuf[pl.ds(xo + j * 16, 16)])

            lax.cond(bvec[0] == bvec[15], _uniform, _mixed)
            return 0
        lax.fori_loop(0, _CHUNK // 16, _grp, 0)

    _x_pipeline(x_hbm, x_v0, x_v1, base, semx0, semx1, _process)

    def _sred(i, _):
        rows = (iota + i * 16) * 16
        acc = jnp.zeros((16,), jnp.float32)
        for j in range(16):
            acc = acc + plsc.load_gather(s16_v, [rows + j])
        sks_v[pl.ds(i * 16, 16)] = acc
        return 0
    lax.fori_loop(0, _G // 16, _sred, 0)

    pltpu.sync_copy(sks_v, sk_hbm.at[wid])
    pltpu.sync_copy(r_v.at[pl.ds(0, _G)], rk_hbm.at[wid])


def _lstm(q_star, h_prev, c_prev, wih, whh, bias):
    gates = (lax.dot_general(q_star, wih, (((1,), (1,)), ((), ())),
                             preferred_element_type=jnp.float32) +
             lax.dot_general(h_prev, whh, (((1,), (1,)), ((), ())),
                             preferred_element_type=jnp.float32) + bias)
    i_g = jax.nn.sigmoid(gates[:, 0 * _IN:1 * _IN])
    f_g = jax.nn.sigmoid(gates[:, 1 * _IN:2 * _IN])
    g_g = jnp.tanh(gates[:, 2 * _IN:3 * _IN])
    o_g = jax.nn.sigmoid(gates[:, 3 * _IN:4 * _IN])
    c_new = f_g * c_prev + i_g * g_g
    h_new = o_g * jnp.tanh(c_new)
    return h_new, c_new


def _tc_init_kernel(bias_ref, h_ref, c_ref, q_ref):
    gb = bias_ref[...]
    i_g = jax.nn.sigmoid(gb[:, 0 * _IN:1 * _IN])
    g_g = jnp.tanh(gb[:, 2 * _IN:3 * _IN])
    o_g = jax.nn.sigmoid(gb[:, 3 * _IN:4 * _IN])
    c0 = i_g * g_g
    h0 = o_g * jnp.tanh(c0)
    h_ref[...] = jnp.broadcast_to(h0, (_G, _IN))
    c_ref[...] = jnp.broadcast_to(c0, (_G, _IN))
    q_ref[...] = jnp.broadcast_to(h0, (_G, _IN))


def _combine_r(racc, sacc):
    recip = 1.0 / (sacc + 1e-16)
    rows = lax.broadcasted_iota(jnp.int32, (_G, _G), 0)
    cols = lax.broadcasted_iota(jnp.int32, (_G, _G), 1)
    s_col = jnp.sum(jnp.where(rows == cols, recip, 0.0), axis=1,
                    keepdims=True)
    return racc * s_col


def _tc_step_kernel(rk_ref, sk_ref, h_ref, c_ref, q_ref, wih_ref, whh_ref,
                    bias_ref, ho_ref, co_ref, qo_ref, racc, sacc):
    i = pl.program_id(0)

    @pl.when(i == 0)
    def _():
        racc[...] = jnp.zeros((_G, _IN), jnp.float32)
        sacc[...] = jnp.zeros((1, _G), jnp.float32)

    racc[...] = racc[...] + rk_ref[0]
    sacc[...] = sacc[...] + sk_ref[0]

    @pl.when(i == _NW - 1)
    def _():
        r = _combine_r(racc[...], sacc[...])
        q_star = jnp.concatenate([q_ref[...], r], axis=1)
        h_new, c_new = _lstm(q_star, h_ref[...], c_ref[...],
                             wih_ref[...], whh_ref[...], bias_ref[...])
        ho_ref[...] = h_new
        co_ref[...] = c_new
        qo_ref[...] = h_new


def _tc_fin_kernel(rk_ref, sk_ref, q_ref, out_ref, racc, sacc):
    i = pl.program_id(0)

    @pl.when(i == 0)
    def _():
        racc[...] = jnp.zeros((_G, _IN), jnp.float32)
        sacc[...] = jnp.zeros((1, _G), jnp.float32)

    racc[...] = racc[...] + rk_ref[0]
    sacc[...] = sacc[...] + sk_ref[0]

    @pl.when(i == _NW - 1)
    def _():
        r = _combine_r(racc[...], sacc[...])
        out_ref[...] = jnp.concatenate([q_ref[...], r], axis=1)


def kernel(x, batch, W_ih, W_hh, b_ih, b_hh):
    n = x.shape[0]
    x_p = jnp.pad(x, ((0, _NPAD - n), (0, 0))).reshape(-1)
    batch_p = jnp.pad(batch, (0, _NPAD - n), constant_values=_G)
    bias = (b_ih + b_hh).reshape(1, 4 * _IN)

    h, c, q = pl.pallas_call(
        _tc_init_kernel,
        out_shape=[jax.ShapeDtypeStruct((_G, _IN), jnp.float32)] * 3,
    )(bias)

    mesh = plsc.VectorSubcoreMesh(core_axis_name="c", subcore_axis_name="s")
    f32 = jnp.float32

    scp = pltpu.CompilerParams(needs_layout_passes=False)
    a_call = pl.kernel(
        _sc_e_kernel,
        out_type=[jax.ShapeDtypeStruct((_NPAD,), f32),
                  jax.ShapeDtypeStruct((_NW, _G), f32)],
        mesh=mesh,
        compiler_params=scp,
        scratch_types=[
            pltpu.VMEM((_GP * _IN,), f32),
            pltpu.VMEM((_CHUNK * _IN,), f32),
            pltpu.VMEM((_CHUNK * _IN,), f32),
            pltpu.VMEM((_NPW,), jnp.int32),
            pltpu.VMEM((_NPW,), f32),
            pltpu.VMEM((_GP * 16,), f32),
            pltpu.VMEM((_G,), f32),
            pltpu.SemaphoreType.DMA,
            pltpu.SemaphoreType.DMA,
        ],
    )

    c_call = pl.kernel(
        _sc_r_kernel,
        out_type=[jax.ShapeDtypeStruct((_NW, _G), f32),
                  jax.ShapeDtypeStruct((_NW, _G, _IN), f32)],
        mesh=mesh,
        compiler_params=scp,
        scratch_types=[
            pltpu.VMEM((_CHUNK * _IN,), f32),
            pltpu.VMEM((_CHUNK * _IN,), f32),
            pltpu.VMEM((_NPW,), jnp.int32),
            pltpu.VMEM((_NPW,), f32),
            pltpu.VMEM((_GP,), f32),
            pltpu.VMEM((_GP * 16,), f32),
            pltpu.VMEM((_GP, _IN), f32),
            pltpu.VMEM((_NW, _G), f32),
            pltpu.VMEM((_G,), f32),
            pltpu.SemaphoreType.DMA,
            pltpu.SemaphoreType.DMA,
        ],
    )

    step_call = pl.pallas_call(
        _tc_step_kernel,
        grid=(_NW,),
        in_specs=[
            pl.BlockSpec((1, _G, _IN), lambda i: (i, 0, 0)),
            pl.BlockSpec((1, 1, _G), lambda i: (i, 0, 0)),
            pl.BlockSpec((_G, _IN), lambda i: (0, 0)),
            pl.BlockSpec((_G, _IN), lambda i: (0, 0)),
            pl.BlockSpec((_G, _IN), lambda i: (0, 0)),
            pl.BlockSpec((4 * _IN, 2 * _IN), lambda i: (0, 0)),
            pl.BlockSpec((4 * _IN, _IN), lambda i: (0, 0)),
            pl.BlockSpec((1, 4 * _IN), lambda i: (0, 0)),
        ],
        out_specs=[pl.BlockSpec((_G, _IN), lambda i: (0, 0))] * 3,
        out_shape=[jax.ShapeDtypeStruct((_G, _IN), f32)] * 3,
        scratch_shapes=[pltpu.VMEM((_G, _IN), f32), pltpu.VMEM((1, _G), f32)],
    )

    fin_call = pl.pallas_call(
        _tc_fin_kernel,
        grid=(_NW,),
        in_specs=[
            pl.BlockSpec((1, _G, _IN), lambda i: (i, 0, 0)),
            pl.BlockSpec((1, 1, _G), lambda i: (i, 0, 0)),
            pl.BlockSpec((_G, _IN), lambda i: (0, 0)),
        ],
        out_specs=pl.BlockSpec((_G, 2 * _IN), lambda i: (0, 0)),
        out_shape=jax.ShapeDtypeStruct((_G, 2 * _IN), f32),
        scratch_shapes=[pltpu.VMEM((_G, _IN), f32), pltpu.VMEM((1, _G), f32)],
    )

    q_flat = None
    out = None
    for t in range(_STEPS):
        q_flat = q.reshape(-1)
        e, mk = a_call(x_p, batch_p, q_flat)
        sk, rk = c_call(x_p, batch_p, e, mk)
        sk3 = sk.reshape(_NW, 1, _G)
        if t < _STEPS - 1:
            h, c, q = step_call(rk, sk3, h, c, q, W_ih, W_hh, bias)
        else:
            out = fin_call(rk, sk3, q)
    return out


# speedup vs baseline: 1.8151x; 1.8151x over previous
import jax
import jax.numpy as jnp
from jax import lax
from jax.experimental import pallas as pl
from jax.experimental.pallas import tpu as pltpu
from jax.experimental.pallas import tpu_sc as plsc

_IN = 128
_STEPS = 3
_G = 512
_GP = 528
_NW = 32
_NPAD = 100352
_NPW = _NPAD // _NW
_CHUNK = 112
_NCH = _NPW // _CHUNK
_NEG = -1e30


def _wid():
    return lax.axis_index("s") * 2 + lax.axis_index("c")


def _x_pipeline(x_hbm, x_v0, x_v1, base, semx0, semx1, process):
    def _issue(k, buf, sem):
        off = (base + k * _CHUNK) * _IN
        pltpu.make_async_copy(
            x_hbm.at[pl.ds(off, _CHUNK * _IN)], buf, sem).start()

    def _wait(buf, sem):
        pltpu.make_async_copy(
            x_hbm.at[pl.ds(0, _CHUNK * _IN)], buf, sem).wait()

    _issue(0, x_v0, semx0)

    def _pair(k, _):
        _issue(2 * k + 1, x_v1, semx1)
        _wait(x_v0, semx0)
        process(2 * k, x_v0)

        @pl.when(k < _NCH // 2 - 1)
        def _():
            _issue(2 * k + 2, x_v0, semx0)

        _wait(x_v1, semx1)
        process(2 * k + 1, x_v1)
        return 0
    lax.fori_loop(0, _NCH // 2, _pair, 0)


def _sc_e_kernel(x_hbm, b_hbm, q_hbm, e_hbm, mk_hbm,
                 q_v, x_v0, x_v1, ids_v, e_v, m16_v, mks_v, semx0, semx1):
    wid = _wid()
    base = wid * _NPW
    iota = lax.iota(jnp.int32, 16)

    pltpu.sync_copy(b_hbm.at[pl.ds(base, _NPW)], ids_v)
    pltpu.sync_copy(q_hbm, q_v.at[pl.ds(0, _G * _IN)])

    def _minit(i, _):
        m16_v[pl.ds(i * 16, 16)] = jnp.full((16,), _NEG, jnp.float32)
        return 0
    lax.fori_loop(0, _GP, _minit, 0)

    def _qzero(i, _):
        q_v[pl.ds(_G * _IN + i * 16, 16)] = jnp.zeros((16,), jnp.float32)
        return 0
    lax.fori_loop(0, (_GP - _G) * _IN // 16, _qzero, 0)

    def _process(kk, xbuf):
        off = kk * _CHUNK

        def _grp(g, _):
            nn = off + g * 16
            bvec = ids_v[pl.ds(nn, 16)]

            def _uniform():
                b0 = bvec[0]
                qregs = [q_v[pl.ds(b0 * _IN + j * 16, 16)]
                         for j in range(_IN // 16)]
                evec = jnp.zeros((16,), jnp.float32)
                for l in range(16):
                    xo = (g * 16 + l) * _IN
                    p = [xbuf[pl.ds(xo + j * 16, 16)] * qregs[j]
                         for j in range(_IN // 16)]
                    t = (((p[0] + p[1]) + (p[2] + p[3])) +
                         ((p[4] + p[5]) + (p[6] + p[7])))
                    evec = jnp.where(iota == l, jnp.sum(t), evec)
                return evec

            def _mixed():
                xb = (iota + g * 16) * _IN
                qb = bvec * _IN

                def _cblk(ci, accs):
                    out = []
                    for u in range(8):
                        cidx = ci * 8 + u
                        xc = plsc.load_gather(xbuf, [xb + cidx])
                        qc = plsc.load_gather(q_v, [qb + cidx])
                        out = out + [accs[u] + xc * qc]
                    return tuple(out)
                accs = lax.fori_loop(
                    0, _IN // 8, _cblk,
                    tuple(jnp.zeros((16,), jnp.float32) for _ in range(8)))
                return (((accs[0] + accs[1]) + (accs[2] + accs[3])) +
                        ((accs[4] + accs[5]) + (accs[6] + accs[7])))

            acc = lax.cond(bvec[0] == bvec[15], _uniform, _mixed)
            e_v[pl.ds(nn, 16)] = acc
            i16 = bvec * 16 + iota
            mg = plsc.load_gather(m16_v, [i16])
            plsc.store_scatter(m16_v, [i16], jnp.maximum(mg, acc))
            return 0
        lax.fori_loop(0, _CHUNK // 16, _grp, 0)

    _x_pipeline(x_hbm, x_v0, x_v1, base, semx0, semx1, _process)

    def _mred(i, _):
        rows = (iota + i * 16) * 16
        acc = jnp.full((16,), _NEG, jnp.float32)
        for j in range(16):
            acc = jnp.maximum(acc, plsc.load_gather(m16_v, [rows + j]))
        mks_v[pl.ds(i * 16, 16)] = acc
        return 0
    lax.fori_loop(0, _G // 16, _mred, 0)

    pltpu.sync_copy(e_v, e_hbm.at[pl.ds(base, _NPW)])
    pltpu.sync_copy(mks_v, mk_hbm.at[wid])


def _sc_r_kernel(x_hbm, b_hbm, e_hbm, mk_hbm, sk_hbm, rk_hbm,
                 x_v0, x_v1, ids_v, e_v, m_v, s16_v, r_v, mk_v, sks_v,
                 semx0, semx1):
    wid = _wid()
    base = wid * _NPW
    iota = lax.iota(jnp.int32, 16)

    pltpu.sync_copy(mk_hbm, mk_v)
    pltpu.sync_copy(b_hbm.at[pl.ds(base, _NPW)], ids_v)
    pltpu.sync_copy(e_hbm.at[pl.ds(base, _NPW)], e_v)

    def _mred(i, _):
        sl = pl.ds(i * 16, 16)

        def _w(w, a):
            return jnp.maximum(a, mk_v[w, sl])
        m_v[sl] = lax.fori_loop(0, _NW, _w,
                                jnp.full((16,), _NEG, jnp.float32))
        return 0
    lax.fori_loop(0, _G // 16, _mred, 0)

    def _mpad(i, _):
        m_v[pl.ds(_G + i * 16, 16)] = jnp.zeros((16,), jnp.float32)
        return 0
    lax.fori_loop(0, (_GP - _G) // 16, _mpad, 0)

    def _szero(i, _):
        s16_v[pl.ds(i * 16, 16)] = jnp.zeros((16,), jnp.float32)
        return 0
    lax.fori_loop(0, _GP, _szero, 0)

    def _rzero(i, _):
        for j in range(_IN // 16):
            r_v[i, pl.ds(j * 16, 16)] = jnp.zeros((16,), jnp.float32)
        return 0
    lax.fori_loop(0, _GP, _rzero, 0)

    def _process(kk, xbuf):
        off = kk * _CHUNK

        def _grp(g, _):
            nn = off + g * 16
            bvec = ids_v[pl.ds(nn, 16)]
            evec = e_v[pl.ds(nn, 16)]
            mg = plsc.load_gather(m_v, [bvec])
            exv = jnp.exp(evec - mg)
            i16 = bvec * 16 + iota
            sg = plsc.load_gather(s16_v, [i16])
            plsc.store_scatter(s16_v, [i16], sg + exv)
            xo0 = g * 16 * _IN

            def _uniform():
                exns = [exv[l] for l in range(16)]
                accs = [jnp.zeros((16,), jnp.float32) for _ in range(8)]
                for l in range(16):
                    for j in range(_IN // 16):
                        accs[j] = accs[j] + exns[l] * xbuf[
                            pl.ds(xo0 + l * _IN + j * 16, 16)]
                bn = bvec[0]
                for j in range(_IN // 16):
                    sl = pl.ds(j * 16, 16)
                    r_v[bn, sl] = r_v[bn, sl] + accs[j]

            def _mixed():
                for l in range(16):
                    bn = bvec[l]
                    exn = exv[l]
                    xo = xo0 + l * _IN
                    for j in range(_IN // 16):
                        sl = pl.ds(j * 16, 16)
                        r_v[bn, sl] = (r_v[bn, sl] +
                                       exn * xbuf[pl.ds(xo + j * 16, 16)])

            lax.cond(bvec[0] == bvec[15], _uniform, _mixed)
            return 0
        lax.fori_loop(0, _CHUNK // 16, _grp, 0)

    _x_pipeline(x_hbm, x_v0, x_v1, base, semx0, semx1, _process)

    def _sred(i, _):
        rows = (iota + i * 16) * 16
        acc = jnp.zeros((16,), jnp.float32)
        for j in range(16):
            acc = acc + plsc.load_gather(s16_v, [rows + j])
        sks_v[pl.ds(i * 16, 16)] = acc
        return 0
    lax.fori_loop(0, _G // 16, _sred, 0)

    pltpu.sync_copy(sks_v, sk_hbm.at[wid])
    pltpu.sync_copy(r_v.at[pl.ds(0, _G)], rk_hbm.at[wid])


def _lstm(q_star, h_prev, c_prev, wih, whh, bias):
    gates = (lax.dot_general(q_star, wih, (((1,), (1,)), ((), ())),
                             preferred_element_type=jnp.float32) +
             lax.dot_general(h_prev, whh, (((1,), (1,)), ((), ())),
                             preferred_element_type=jnp.float32) + bias)
    i_g = jax.nn.sigmoid(gates[:, 0 * _IN:1 * _IN])
    f_g = jax.nn.sigmoid(gates[:, 1 * _IN:2 * _IN])
    g_g = jnp.tanh(gates[:, 2 * _IN:3 * _IN])
    o_g = jax.nn.sigmoid(gates[:, 3 * _IN:4 * _IN])
    c_new = f_g * c_prev + i_g * g_g
    h_new = o_g * jnp.tanh(c_new)
    return h_new, c_new


def _tc_init_kernel(bias_ref, h_ref, c_ref, q_ref):
    gb = bias_ref[...]
    i_g = jax.nn.sigmoid(gb[:, 0 * _IN:1 * _IN])
    g_g = jnp.tanh(gb[:, 2 * _IN:3 * _IN])
    o_g = jax.nn.sigmoid(gb[:, 3 * _IN:4 * _IN])
    c0 = i_g * g_g
    h0 = o_g * jnp.tanh(c0)
    h_ref[...] = jnp.broadcast_to(h0, (_G, _IN))
    c_ref[...] = jnp.broadcast_to(c0, (_G, _IN))
    q_ref[...] = jnp.broadcast_to(h0, (_G, _IN))


def _combine_r(racc, sacc):
    recip = 1.0 / (sacc + 1e-16)
    rows = lax.broadcasted_iota(jnp.int32, (_G, _G), 0)
    cols = lax.broadcasted_iota(jnp.int32, (_G, _G), 1)
    s_col = jnp.sum(jnp.where(rows == cols, recip, 0.0), axis=1,
                    keepdims=True)
    return racc * s_col


def _tc_step_kernel(rk_ref, sk_ref, h_ref, c_ref, q_ref, wih_ref, whh_ref,
                    bias_ref, ho_ref, co_ref, qo_ref, racc, sacc):
    i = pl.program_id(0)

    @pl.when(i == 0)
    def _():
        racc[...] = jnp.zeros((_G, _IN), jnp.float32)
        sacc[...] = jnp.zeros((1, _G), jnp.float32)

    racc[...] = racc[...] + rk_ref[0]
    sacc[...] = sacc[...] + sk_ref[0]

    @pl.when(i == _NW - 1)
    def _():
        r = _combine_r(racc[...], sacc[...])
        q_star = jnp.concatenate([q_ref[...], r], axis=1)
        h_new, c_new = _lstm(q_star, h_ref[...], c_ref[...],
                             wih_ref[...], whh_ref[...], bias_ref[...])
        ho_ref[...] = h_new
        co_ref[...] = c_new
        qo_ref[...] = h_new


def _tc_fin_kernel(rk_ref, sk_ref, q_ref, out_ref, racc, sacc):
    i = pl.program_id(0)

    @pl.when(i == 0)
    def _():
        racc[...] = jnp.zeros((_G, _IN), jnp.float32)
        sacc[...] = jnp.zeros((1, _G), jnp.float32)

    racc[...] = racc[...] + rk_ref[0]
    sacc[...] = sacc[...] + sk_ref[0]

    @pl.when(i == _NW - 1)
    def _():
        r = _combine_r(racc[...], sacc[...])
        out_ref[...] = jnp.concatenate([q_ref[...], r], axis=1)


def kernel(x, batch, W_ih, W_hh, b_ih, b_hh):
    n = x.shape[0]
    x_p = jnp.pad(x, ((0, _NPAD - n), (0, 0))).reshape(-1)
    batch_p = jnp.pad(batch, (0, _NPAD - n), constant_values=_G)
    bias = (b_ih + b_hh).reshape(1, 4 * _IN)

    h, c, q = pl.pallas_call(
        _tc_init_kernel,
        out_shape=[jax.ShapeDtypeStruct((_G, _IN), jnp.float32)] * 3,
    )(bias)

    mesh = plsc.VectorSubcoreMesh(core_axis_name="c", subcore_axis_name="s")
    f32 = jnp.float32

    scp = pltpu.CompilerParams(needs_layout_passes=False)
    a_call = pl.kernel(
        _sc_e_kernel,
        out_type=[jax.ShapeDtypeStruct((_NPAD,), f32),
                  jax.ShapeDtypeStruct((_NW, _G), f32)],
        mesh=mesh,
        compiler_params=scp,
        scratch_types=[
            pltpu.VMEM((_GP * _IN,), f32),
            pltpu.VMEM((_CHUNK * _IN,), f32),
            pltpu.VMEM((_CHUNK * _IN,), f32),
            pltpu.VMEM((_NPW,), jnp.int32),
            pltpu.VMEM((_NPW,), f32),
            pltpu.VMEM((_GP * 16,), f32),
            pltpu.VMEM((_G,), f32),
            pltpu.SemaphoreType.DMA,
            pltpu.SemaphoreType.DMA,
        ],
    )

    c_call = pl.kernel(
        _sc_r_kernel,
        out_type=[jax.ShapeDtypeStruct((_NW, _G), f32),
                  jax.ShapeDtypeStruct((_NW, _G, _IN), f32)],
        mesh=mesh,
        compiler_params=scp,
        scratch_types=[
            pltpu.VMEM((_CHUNK * _IN,), f32),
            pltpu.VMEM((_CHUNK * _IN,), f32),
            pltpu.VMEM((_NPW,), jnp.int32),
            pltpu.VMEM((_NPW,), f32),
            pltpu.VMEM((_GP,), f32),
            pltpu.VMEM((_GP * 16,), f32),
            pltpu.VMEM((_GP, _IN), f32),
            pltpu.VMEM((_NW, _G), f32),
            pltpu.VMEM((_G,), f32),
            pltpu.SemaphoreType.DMA,
            pltpu.SemaphoreType.DMA,
        ],
    )

    step_call = pl.pallas_call(
        _tc_step_kernel,
        grid=(_NW,),
        in_specs=[
            pl.BlockSpec((1, _G, _IN), lambda i: (i, 0, 0)),
            pl.BlockSpec((1, 1, _G), lambda i: (i, 0, 0)),
            pl.BlockSpec((_G, _IN), lambda i: (0, 0)),
            pl.BlockSpec((_G, _IN), lambda i: (0, 0)),
            pl.BlockSpec((_G, _IN), lambda i: (0, 0)),
            pl.BlockSpec((4 * _IN, 2 * _IN), lambda i: (0, 0)),
            pl.BlockSpec((4 * _IN, _IN), lambda i: (0, 0)),
            pl.BlockSpec((1, 4 * _IN), lambda i: (0, 0)),
        ],
        out_specs=[pl.BlockSpec((_G, _IN), lambda i: (0, 0))] * 3,
        out_shape=[jax.ShapeDtypeStruct((_G, _IN), f32)] * 3,
        scratch_shapes=[pltpu.VMEM((_G, _IN), f32), pltpu.VMEM((1, _G), f32)],
    )

    fin_call = pl.pallas_call(
        _tc_fin_kernel,
        grid=(_NW,),
        in_specs=[
            pl.BlockSpec((1, _G, _IN), lambda i: (i, 0, 0)),
            pl.BlockSpec((1, 1, _G), lambda i: (i, 0, 0)),
            pl.BlockSpec((_G, _IN), lambda i: (0, 0)),
        ],
        out_specs=pl.BlockSpec((_G, 2 * _IN), lambda i: (0, 0)),
        out_shape=jax.ShapeDtypeStruct((_G, 2 * _IN), f32),
        scratch_shapes=[pltpu.VMEM((_G, _IN), f32), pltpu.VMEM((1, _G), f32)],
    )

    q_flat = None
    out = None
    for t in range(_STEPS):
        q_flat = q.reshape(-1)
        e, mk = a_call(x_p, batch_p, q_flat)
        sk, rk = c_call(x_p, batch_p, e, mk)
        sk3 = sk.reshape(_NW, 1, _G)
        if t < _STEPS - 1:
            h, c, q = step_call(rk, sk3, h, c, q, W_ih, W_hh, bias)
        else:
            out = fin_call(rk, sk3, q)
    return out
